# baseline (speedup 1.0000x reference)
import functools

import jax
import jax.numpy as jnp
from jax.experimental import pallas as pl

_SIGMA = 10.0
_R2 = (_SIGMA * 3.0) ** 2
_K = 64
_NPH = 20
_NCTRL = 500
_CPAD = 512
_BL = 8192
_NITER = 11


def _rbf_block(a_ref, p_ref, dispT_ref, out_ref):
    p = jnp.transpose(p_ref[...])
    x = p[0:1, :]
    y = p[1:2, :]
    z = p[2:3, :]
    p2 = x * x + y * y + z * z
    ones = jnp.ones_like(p2)
    zeros3 = jnp.zeros((3, p.shape[1]), dtype=p.dtype)
    paug = jnp.concatenate([p[0:3, :], ones, p2, zeros3], axis=0)

    a = a_ref[...]
    d2 = jax.lax.dot_general(
        a, paug, (((1,), (0,)), ((), ())),
        precision=jax.lax.Precision.HIGHEST,
        preferred_element_type=jnp.float32)

    lo = jnp.zeros_like(p2)
    hi = jnp.full_like(p2, _R2 + 1.0)
    tf = hi
    found = jnp.zeros_like(p2, dtype=jnp.bool_)
    kf = jnp.float32(_K)
    for _ in range(_NITER):
        mid = 0.5 * (lo + hi)
        cnt = jnp.sum((d2 <= mid).astype(jnp.float32), axis=0,
                      keepdims=True)
        eq = jnp.logical_and(cnt == kf, jnp.logical_not(found))
        tf = jnp.where(eq, mid, tf)
        found = jnp.logical_or(found, eq)
        ge = cnt >= kf
        hi = jnp.where(ge, mid, hi)
        lo = jnp.where(ge, lo, mid)

    thresh = jnp.minimum(jnp.where(found, tf, hi), _R2)
    w = jnp.exp(d2 * jnp.float32(-1.0 / (2.0 * _SIGMA * _SIGMA)))
    w = jnp.where(d2 <= thresh, w, jnp.float32(0.0))

    wsum = jnp.sum(w, axis=0, keepdims=True)
    wsum = jnp.where(wsum < 1e-5, jnp.float32(1.0), wsum)

    dispT = dispT_ref[...]
    acc = jax.lax.dot_general(
        dispT, w, (((1,), (0,)), ((), ())),
        precision=jax.lax.Precision.DEFAULT,
        preferred_element_type=jnp.float32)
    out_ref[...] = jnp.transpose(acc / wsum)


def _run_shard(pts, ctrl_pts, ctrl_disps, phase):
    t = phase[0] * _NPH
    i0 = jnp.clip(jnp.floor(t).astype(jnp.int32), 0, _NPH - 1)
    i1 = jnp.clip(jnp.ceil(t).astype(jnp.int32), 0, _NPH - 1)
    wp = t - jnp.floor(t)
    disp = wp * jnp.take(ctrl_disps, i1, axis=0) \
        + (1.0 - wp) * jnp.take(ctrl_disps, i0, axis=0)

    c2 = jnp.sum(ctrl_pts * ctrl_pts, axis=1, keepdims=True)
    a = jnp.zeros((_CPAD, 8), dtype=jnp.float32)
    a = a.at[:_NCTRL, 0:3].set(-2.0 * ctrl_pts)
    a = a.at[:_NCTRL, 3:4].set(c2)
    a = a.at[_NCTRL:, 3].set(1e9)
    a = a.at[:, 4].set(1.0)

    dispT = jnp.zeros((8, _CPAD), dtype=jnp.float32)
    dispT = dispT.at[0:3, :_NCTRL].set(disp.T)

    n = pts.shape[0]
    p8 = jnp.pad(pts, ((0, 0), (0, 5)))
    out8 = pl.pallas_call(
        _rbf_block,
        grid=(n // _BL,),
        in_specs=[
            pl.BlockSpec((_CPAD, 8), lambda j: (0, 0)),
            pl.BlockSpec((_BL, 8), lambda j: (j, 0)),
            pl.BlockSpec((8, _CPAD), lambda j: (0, 0)),
        ],
        out_specs=pl.BlockSpec((_BL, 8), lambda j: (j, 0)),
        out_shape=jax.ShapeDtypeStruct((n, 8), jnp.float32),
    )(a, p8, dispT)
    return out8[:, 0:3]


@jax.jit
def kernel(points, ctrl_pts, ctrl_disps, phase):
    return _run_shard(points, ctrl_pts, ctrl_disps, phase).astype(jnp.float32)


# speedup vs baseline: 1.2051x; 1.2051x over previous
import functools

import jax
import jax.numpy as jnp
from jax.experimental import pallas as pl

_SIGMA = 10.0
_R2 = (_SIGMA * 3.0) ** 2
_K = 64
_NPH = 20
_NCTRL = 500
_CPAD = 512
_BL = 8192
_NITER = 11


def _rbf_block(a_ref, p_ref, dispT_ref, out_ref):
    p = p_ref[...]
    x = p[0:1, :]
    y = p[1:2, :]
    z = p[2:3, :]
    p2 = x * x + y * y + z * z
    ones = jnp.ones_like(p2)
    zeros3 = jnp.zeros((3, p.shape[1]), dtype=p.dtype)
    paug = jnp.concatenate([p[0:3, :], ones, p2, zeros3], axis=0)

    a = a_ref[...]
    d2 = jax.lax.dot_general(
        a, paug, (((1,), (0,)), ((), ())),
        precision=jax.lax.Precision.HIGHEST,
        preferred_element_type=jnp.float32)

    lo = jnp.zeros_like(p2)
    hi = jnp.full_like(p2, _R2 + 1.0)
    tf = hi
    found = jnp.zeros_like(p2, dtype=jnp.bool_)
    kf = jnp.float32(_K)
    for _ in range(_NITER):
        mid = 0.5 * (lo + hi)
        cnt = jnp.sum((d2 <= mid).astype(jnp.float32), axis=0,
                      keepdims=True)
        eq = jnp.logical_and(cnt == kf, jnp.logical_not(found))
        tf = jnp.where(eq, mid, tf)
        found = jnp.logical_or(found, eq)
        ge = cnt >= kf
        hi = jnp.where(ge, mid, hi)
        lo = jnp.where(ge, lo, mid)

    thresh = jnp.minimum(jnp.where(found, tf, hi), _R2)
    w = jnp.exp(d2 * jnp.float32(-1.0 / (2.0 * _SIGMA * _SIGMA)))
    w = jnp.where(d2 <= thresh, w, jnp.float32(0.0))

    wsum = jnp.sum(w, axis=0, keepdims=True)
    wsum = jnp.where(wsum < 1e-5, jnp.float32(1.0), wsum)

    dispT = dispT_ref[...]
    acc = jax.lax.dot_general(
        dispT, w, (((1,), (0,)), ((), ())),
        precision=jax.lax.Precision.DEFAULT,
        preferred_element_type=jnp.float32)
    out_ref[...] = acc / wsum


def _run_shard(pts, ctrl_pts, ctrl_disps, phase):
    t = phase[0] * _NPH
    i0 = jnp.clip(jnp.floor(t).astype(jnp.int32), 0, _NPH - 1)
    i1 = jnp.clip(jnp.ceil(t).astype(jnp.int32), 0, _NPH - 1)
    wp = t - jnp.floor(t)
    disp = wp * jnp.take(ctrl_disps, i1, axis=0) \
        + (1.0 - wp) * jnp.take(ctrl_disps, i0, axis=0)

    c2 = jnp.sum(ctrl_pts * ctrl_pts, axis=1, keepdims=True)
    a = jnp.zeros((_CPAD, 8), dtype=jnp.float32)
    a = a.at[:_NCTRL, 0:3].set(-2.0 * ctrl_pts)
    a = a.at[:_NCTRL, 3:4].set(c2)
    a = a.at[_NCTRL:, 3].set(1e9)
    a = a.at[:, 4].set(1.0)

    dispT = jnp.zeros((8, _CPAD), dtype=jnp.float32)
    dispT = dispT.at[0:3, :_NCTRL].set(disp.T)

    n = pts.shape[0]
    pT = jnp.zeros((8, n), jnp.float32)
    pT = pT.at[0:3, :].set(pts.T)
    outT = pl.pallas_call(
        _rbf_block,
        grid=(n // _BL,),
        in_specs=[
            pl.BlockSpec((_CPAD, 8), lambda j: (0, 0)),
            pl.BlockSpec((8, _BL), lambda j: (0, j)),
            pl.BlockSpec((8, _CPAD), lambda j: (0, 0)),
        ],
        out_specs=pl.BlockSpec((8, _BL), lambda j: (0, j)),
        out_shape=jax.ShapeDtypeStruct((8, n), jnp.float32),
    )(a, pT, dispT)
    return outT[0:3, :].T


@jax.jit
def kernel(points, ctrl_pts, ctrl_disps, phase):
    return _run_shard(points, ctrl_pts, ctrl_disps, phase).astype(jnp.float32)
